# baseline (speedup 1.0000x reference)
import functools

import jax
import jax.numpy as jnp
from jax import lax
from jax.experimental import pallas as pl
from jax.experimental.pallas import tpu as pltpu
from jax.experimental.pallas import tpu_sc as plsc

D = 65536
Q = D // 4
B = 256
LANES = 16
NC, NS = 2, 16
NW = NC * NS
PAIR_PER_W = Q // NW
T = 128
ELEMS = T * B
NT = PAIR_PER_W // T
UNROLL = 8


def _cs_body(angle_ref, cs_ref):
    a = angle_ref[0, 0] * 0.5
    c = jnp.cos(a)
    s = jnp.sin(a)
    row = jnp.ones((1, LANES), jnp.float32)
    cs_ref[...] = jnp.concatenate([c * row, -s * row], axis=0)


def _compute_cs(angle):
    return pl.pallas_call(
        _cs_body,
        in_specs=[pl.BlockSpec(memory_space=pltpu.SMEM)],
        out_specs=pl.BlockSpec(memory_space=pltpu.VMEM),
        out_shape=jax.ShapeDtypeStruct((2, LANES), jnp.float32),
    )(angle.reshape(1, 1).astype(jnp.float32))


@functools.partial(
    pl.kernel,
    out_type=jax.ShapeDtypeStruct((2 * Q * B,), jnp.float32),
    mesh=plsc.VectorSubcoreMesh(core_axis_name="c", subcore_axis_name="s"),
    scratch_types=[
        pltpu.VMEM((ELEMS,), jnp.float32),
        pltpu.VMEM((ELEMS,), jnp.float32),
        pltpu.VMEM((2, LANES), jnp.float32),
    ],
)
def _sc_mix(x_hbm, cs_hbm, out_hbm, a_v, b_v, cs_v):
    wid = lax.axis_index("s") * NC + lax.axis_index("c")

    pltpu.sync_copy(cs_hbm, cs_v)
    cvec = cs_v[0, :]
    nsvec = cs_v[1, :]

    in2 = (2 * Q + wid * PAIR_PER_W) * B
    in3 = (3 * Q + wid * PAIR_PER_W) * B
    o2 = (wid * PAIR_PER_W) * B
    o3 = (Q + wid * PAIR_PER_W) * B
    for t in range(NT):
        off = t * ELEMS
        pltpu.sync_copy(x_hbm.at[pl.ds(in2 + off, ELEMS)], a_v)
        pltpu.sync_copy(x_hbm.at[pl.ds(in3 + off, ELEMS)], b_v)

        def mix(i, carry):
            base = i * (LANES * UNROLL)
            for u in range(UNROLL):
                sl = pl.ds(base + u * LANES, LANES)
                av = a_v[sl]
                bv = b_v[sl]
                a_v[sl] = cvec * av + nsvec * bv
                b_v[sl] = cvec * bv + nsvec * av
            return carry

        lax.fori_loop(0, ELEMS // (LANES * UNROLL), mix, 0)
        pltpu.sync_copy(a_v, out_hbm.at[pl.ds(o2 + off, ELEMS)])
        pltpu.sync_copy(b_v, out_hbm.at[pl.ds(o3 + off, ELEMS)])


def kernel(x, angle):
    cs = _compute_cs(angle)
    mixed = _sc_mix(x.reshape(-1), cs)
    y = jnp.concatenate([x[: 2 * Q].reshape(-1), mixed])
    return y.astype(jnp.complex64).reshape(D, B)


# speedup vs baseline: 1.0015x; 1.0015x over previous
import functools

import jax
import jax.numpy as jnp
from jax import lax
from jax.experimental import pallas as pl
from jax.experimental.pallas import tpu as pltpu
from jax.experimental.pallas import tpu_sc as plsc

D = 65536
Q = D // 4
B = 256
LANES = 16
NC, NS = 2, 16
NW = NC * NS
PAIR_PER_W = Q // NW
T = 128
ELEMS = T * B
NT = PAIR_PER_W // T
UNROLL = 8


def _cs_body(angle_ref, cs_ref):
    a = angle_ref[0, 0] * 0.5
    c = jnp.cos(a)
    s = jnp.sin(a)
    row = jnp.ones((1, LANES), jnp.float32)
    cs_ref[...] = jnp.concatenate([c * row, -s * row], axis=0)


def _compute_cs(angle):
    return pl.pallas_call(
        _cs_body,
        in_specs=[pl.BlockSpec(memory_space=pltpu.SMEM)],
        out_specs=pl.BlockSpec(memory_space=pltpu.VMEM),
        out_shape=jax.ShapeDtypeStruct((2, LANES), jnp.float32),
    )(angle.reshape(1, 1).astype(jnp.float32))


@functools.partial(
    pl.kernel,
    out_type=jax.ShapeDtypeStruct((2 * Q * B,), jnp.float32),
    mesh=plsc.VectorSubcoreMesh(core_axis_name="c", subcore_axis_name="s"),
    scratch_types=[
        pltpu.VMEM((ELEMS,), jnp.float32),
        pltpu.VMEM((ELEMS,), jnp.float32),
        pltpu.VMEM((2, LANES), jnp.float32),
    ],
)
def _sc_mix(x_hbm, cs_hbm, out_hbm, a_v, b_v, cs_v):
    wid = lax.axis_index("s") * NC + lax.axis_index("c")

    pltpu.sync_copy(cs_hbm, cs_v)
    cvec = cs_v[0, :]
    nsvec = cs_v[1, :]

    in2 = (2 * Q + wid * PAIR_PER_W) * B
    in3 = (3 * Q + wid * PAIR_PER_W) * B
    o2 = (wid * PAIR_PER_W) * B
    o3 = (Q + wid * PAIR_PER_W) * B
    for t in range(NT):
        off = t * ELEMS
        pltpu.sync_copy(x_hbm.at[pl.ds(in2 + off, ELEMS)], a_v)
        pltpu.sync_copy(x_hbm.at[pl.ds(in3 + off, ELEMS)], b_v)

        def mix(i, carry):
            base = i * (LANES * UNROLL)
            for u in range(UNROLL):
                sl = pl.ds(base + u * LANES, LANES)
                av = a_v[sl]
                bv = b_v[sl]
                a_v[sl] = cvec * av + nsvec * bv
                b_v[sl] = cvec * bv + nsvec * av
            return carry

        lax.fori_loop(0, ELEMS // (LANES * UNROLL), mix, 0)
        pltpu.sync_copy(a_v, out_hbm.at[pl.ds(o2 + off, ELEMS)])
        pltpu.sync_copy(b_v, out_hbm.at[pl.ds(o3 + off, ELEMS)])


def kernel(x, angle):
    cs = _compute_cs(angle)
    mixed = _sc_mix(x.reshape(-1), cs)
    top = x[: 2 * Q].astype(jnp.complex64)
    bot = mixed.reshape(2 * Q, B).astype(jnp.complex64)
    return jnp.concatenate([top, bot], axis=0)
